# baseline (device time: 183942 ns/iter reference)
import jax
import jax.numpy as jnp
from jax import lax
from jax.experimental import pallas as pl
from jax.experimental.pallas import tpu as pltpu

N_DEV = 4
B = 2
SQ = 512
SKV = 512
HQ = 32
DH = 64
HG = HQ // N_DEV
D_MODEL = 768
D_HEADS = HQ * DH
G_COLS = HG * DH


def kernel(x, Wq, K_ext, V_ext, Wo):
    K2 = K_ext.reshape(B, SKV, D_HEADS)
    V2 = V_ext.reshape(B, SKV, D_HEADS)

    def body(x_ref, wq_ref, k_ref, v_ref, wo_ref, out_ref,
             wqb, wob, qsend, qrecv, osend, orecv, csem):
        me = lax.axis_index("i")
        left = (me - 1) % N_DEV
        right = (me + 1) % N_DEV

        bsem = pltpu.get_barrier_semaphore()
        pl.semaphore_signal(bsem, inc=1, device_id=(left,),
                            device_id_type=pl.DeviceIdType.MESH)
        pl.semaphore_signal(bsem, inc=1, device_id=(right,),
                            device_id_type=pl.DeviceIdType.MESH)
        pl.semaphore_wait(bsem, 2)

        cq = pltpu.make_async_copy(wq_ref, wqb.at[me], csem.at[0])
        co = pltpu.make_async_copy(wo_ref, wob.at[me], csem.at[1])
        cq.start()
        co.start()
        cq.wait()
        co.wait()

        for h in range(N_DEV - 1):
            o = (me - h) % N_DEV
            rq = pltpu.make_async_remote_copy(
                src_ref=wqb.at[o], dst_ref=wqb.at[o],
                send_sem=qsend.at[h], recv_sem=qrecv.at[h],
                device_id=(right,), device_id_type=pl.DeviceIdType.MESH)
            ro = pltpu.make_async_remote_copy(
                src_ref=wob.at[o], dst_ref=wob.at[o],
                send_sem=osend.at[h], recv_sem=orecv.at[h],
                device_id=(right,), device_id_type=pl.DeviceIdType.MESH)
            rq.start()
            ro.start()
            rq.wait()
            ro.wait()

        li = lax.broadcasted_iota(jnp.int32, (SQ, SKV), 0)
        kj = lax.broadcasted_iota(jnp.int32, (SQ, SKV), 1)
        qb = me * (SQ // 64) + li // 64
        kb = kj // 64
        mask = jnp.logical_or(qb == kb, qb % 4 == kb % 4)

        for b in range(B):
            xb = x_ref[b]
            acc = jnp.zeros((SQ, D_MODEL), jnp.float32)
            for g in range(N_DEV):
                q = lax.dot_general(
                    xb, wqb[g], (((1,), (0,)), ((), ())),
                    preferred_element_type=jnp.float32)
                ctx_parts = []
                for hh in range(HG):
                    c0 = (g * HG + hh) * DH
                    qh = q[:, hh * DH:(hh + 1) * DH]
                    kh = k_ref[b, :, c0:c0 + DH]
                    s = lax.dot_general(
                        qh, kh, (((1,), (1,)), ((), ())),
                        preferred_element_type=jnp.float32)
                    s = jnp.where(mask, s * 0.125, -1e9)
                    m = jnp.max(s, axis=1, keepdims=True)
                    e = jnp.exp(s - m)
                    w = e / jnp.sum(e, axis=1, keepdims=True)
                    vh = v_ref[b, :, c0:c0 + DH]
                    ctx_parts.append(lax.dot_general(
                        w, vh, (((1,), (0,)), ((), ())),
                        preferred_element_type=jnp.float32))
                ctx = jnp.concatenate(ctx_parts, axis=1)
                acc = acc + lax.dot_general(
                    ctx, wob[g], (((1,), (0,)), ((), ())),
                    preferred_element_type=jnp.float32)
            out_ref[b] = acc

    return pl.pallas_call(
        body,
        out_shape=jax.ShapeDtypeStruct((B, SQ, D_MODEL), jnp.float32),
        in_specs=[pl.BlockSpec(memory_space=pltpu.VMEM)] * 5,
        out_specs=pl.BlockSpec(memory_space=pltpu.VMEM),
        scratch_shapes=[
            pltpu.VMEM((N_DEV, D_MODEL, G_COLS), jnp.float32),
            pltpu.VMEM((N_DEV, G_COLS, D_MODEL), jnp.float32),
            pltpu.SemaphoreType.DMA((N_DEV - 1,)),
            pltpu.SemaphoreType.DMA((N_DEV - 1,)),
            pltpu.SemaphoreType.DMA((N_DEV - 1,)),
            pltpu.SemaphoreType.DMA((N_DEV - 1,)),
            pltpu.SemaphoreType.DMA((2,)),
        ],
        compiler_params=pltpu.CompilerParams(collective_id=0),
    )(x, Wq, K2, V2, Wo)


# device time: 142977 ns/iter; 1.2865x vs baseline; 1.2865x over previous
import jax
import jax.numpy as jnp
from jax import lax
from jax.experimental import pallas as pl
from jax.experimental.pallas import tpu as pltpu

N_DEV = 4
B = 2
SQ = 512
SKV = 512
HQ = 32
DH = 64
HG = HQ // N_DEV
D_MODEL = 768
D_HEADS = HQ * DH
G_COLS = HG * DH


def kernel(x, Wq, K_ext, V_ext, Wo):
    K2 = K_ext.reshape(B, SKV, D_HEADS)
    V2 = V_ext.reshape(B, SKV, D_HEADS)

    def body(x_ref, wq_ref, k_ref, v_ref, wo_ref, out_ref,
             wqb, wob, qsend, qrecv, osend, orecv, csem):
        me = lax.axis_index("i")
        left = (me - 1) % N_DEV
        right = (me + 1) % N_DEV

        bsem = pltpu.get_barrier_semaphore()
        pl.semaphore_signal(bsem, inc=1, device_id=(left,),
                            device_id_type=pl.DeviceIdType.MESH)
        pl.semaphore_signal(bsem, inc=1, device_id=(right,),
                            device_id_type=pl.DeviceIdType.MESH)
        pl.semaphore_wait(bsem, 2)

        cq = pltpu.make_async_copy(wq_ref, wqb.at[0], csem.at[0])
        co = pltpu.make_async_copy(wo_ref, wob.at[0], csem.at[1])
        cq.start()
        co.start()
        cq.wait()
        co.wait()

        li = lax.broadcasted_iota(jnp.int32, (SQ, SKV), 0)
        kj = lax.broadcasted_iota(jnp.int32, (SQ, SKV), 1)
        qb = me * (SQ // 64) + li // 64
        kb = kj // 64
        mask = jnp.logical_or(qb == kb, qb % 4 == kb % 4)

        xs = [x_ref[b] for b in range(B)]
        accs = [jnp.zeros((SQ, D_MODEL), jnp.float32) for _ in range(B)]

        def compute_group(slot, origin):
            col0 = origin * G_COLS
            for b in range(B):
                q = lax.dot_general(
                    xs[b], wqb[slot], (((1,), (0,)), ((), ())),
                    preferred_element_type=jnp.float32)
                kg = k_ref[b, :, pl.ds(col0, G_COLS)]
                vg = v_ref[b, :, pl.ds(col0, G_COLS)]
                ctx_parts = []
                for hh in range(HG):
                    qh = q[:, hh * DH:(hh + 1) * DH]
                    kh = kg[:, hh * DH:(hh + 1) * DH]
                    s = lax.dot_general(
                        qh, kh, (((1,), (1,)), ((), ())),
                        preferred_element_type=jnp.float32)
                    s = jnp.where(mask, s * 0.125, -1e9)
                    m = jnp.max(s, axis=1, keepdims=True)
                    e = jnp.exp(s - m)
                    w = e / jnp.sum(e, axis=1, keepdims=True)
                    ctx_parts.append(lax.dot_general(
                        w, vg[:, hh * DH:(hh + 1) * DH],
                        (((1,), (0,)), ((), ())),
                        preferred_element_type=jnp.float32))
                ctx = jnp.concatenate(ctx_parts, axis=1)
                accs[b] = accs[b] + lax.dot_general(
                    ctx, wob[slot], (((1,), (0,)), ((), ())),
                    preferred_element_type=jnp.float32)

        for h in range(N_DEV - 1):
            rq = pltpu.make_async_remote_copy(
                src_ref=wqb.at[h], dst_ref=wqb.at[h + 1],
                send_sem=qsend.at[h], recv_sem=qrecv.at[h],
                device_id=(right,), device_id_type=pl.DeviceIdType.MESH)
            ro = pltpu.make_async_remote_copy(
                src_ref=wob.at[h], dst_ref=wob.at[h + 1],
                send_sem=osend.at[h], recv_sem=orecv.at[h],
                device_id=(right,), device_id_type=pl.DeviceIdType.MESH)
            rq.start()
            ro.start()
            compute_group(h, (me - h) % N_DEV)
            rq.wait()
            ro.wait()
        compute_group(N_DEV - 1, (me - (N_DEV - 1)) % N_DEV)

        for b in range(B):
            out_ref[b] = accs[b]

    return pl.pallas_call(
        body,
        out_shape=jax.ShapeDtypeStruct((B, SQ, D_MODEL), jnp.float32),
        in_specs=[pl.BlockSpec(memory_space=pltpu.VMEM)] * 5,
        out_specs=pl.BlockSpec(memory_space=pltpu.VMEM),
        scratch_shapes=[
            pltpu.VMEM((N_DEV, D_MODEL, G_COLS), jnp.float32),
            pltpu.VMEM((N_DEV, G_COLS, D_MODEL), jnp.float32),
            pltpu.SemaphoreType.DMA((N_DEV - 1,)),
            pltpu.SemaphoreType.DMA((N_DEV - 1,)),
            pltpu.SemaphoreType.DMA((N_DEV - 1,)),
            pltpu.SemaphoreType.DMA((N_DEV - 1,)),
            pltpu.SemaphoreType.DMA((2,)),
        ],
        compiler_params=pltpu.CompilerParams(collective_id=0),
    )(x, Wq, K2, V2, Wo)


# device time: 86851 ns/iter; 2.1179x vs baseline; 1.6462x over previous
import jax
import jax.numpy as jnp
from jax import lax
from jax.experimental import pallas as pl
from jax.experimental.pallas import tpu as pltpu

N_DEV = 4
B = 2
SQ = 512
SKV = 512
HQ = 32
DH = 64
HG = HQ // N_DEV
D_MODEL = 768
D_HEADS = HQ * DH
G_COLS = HG * DH


def kernel(x, Wq, K_ext, V_ext, Wo):
    bf = jnp.bfloat16
    K2 = K_ext.reshape(B, SKV, D_HEADS).astype(bf)
    V2 = V_ext.reshape(B, SKV, D_HEADS).astype(bf)
    x = x.astype(bf)
    Wq = Wq.astype(bf)
    Wo = Wo.astype(bf)

    def body(x_ref, wq_ref, k_ref, v_ref, wo_ref, out_ref,
             wqb, wob, qsend, qrecv, osend, orecv, csem):
        me = lax.axis_index("i")
        left = (me - 1) % N_DEV
        right = (me + 1) % N_DEV

        bsem = pltpu.get_barrier_semaphore()
        pl.semaphore_signal(bsem, inc=1, device_id=(left,),
                            device_id_type=pl.DeviceIdType.MESH)
        pl.semaphore_signal(bsem, inc=1, device_id=(right,),
                            device_id_type=pl.DeviceIdType.MESH)
        pl.semaphore_wait(bsem, 2)

        cq = pltpu.make_async_copy(wq_ref, wqb.at[0], csem.at[0])
        co = pltpu.make_async_copy(wo_ref, wob.at[0], csem.at[1])
        cq.start()
        co.start()
        cq.wait()
        co.wait()

        li = lax.broadcasted_iota(jnp.int32, (SQ, SKV), 0)
        kj = lax.broadcasted_iota(jnp.int32, (SQ, SKV), 1)
        qb = me * (SQ // 64) + li // 64
        kb = kj // 64
        mask = jnp.logical_or(qb == kb, qb % 4 == kb % 4)

        xs = [x_ref[b] for b in range(B)]
        accs = [jnp.zeros((SQ, D_MODEL), jnp.float32) for _ in range(B)]

        def compute_group(slot, origin):
            col0 = origin * G_COLS
            for b in range(B):
                q = lax.dot_general(
                    xs[b], wqb[slot], (((1,), (0,)), ((), ())),
                    preferred_element_type=jnp.float32).astype(jnp.bfloat16)
                kg = k_ref[b, :, pl.ds(col0, G_COLS)]
                vg = v_ref[b, :, pl.ds(col0, G_COLS)]
                ctx_parts = []
                for hh in range(HG):
                    qh = q[:, hh * DH:(hh + 1) * DH]
                    kh = kg[:, hh * DH:(hh + 1) * DH]
                    s = lax.dot_general(
                        qh, kh, (((1,), (1,)), ((), ())),
                        preferred_element_type=jnp.float32)
                    s = jnp.where(mask, s * 0.125, -1e9)
                    m = jnp.max(s, axis=1, keepdims=True)
                    e = jnp.exp(s - m)
                    w = (e / jnp.sum(e, axis=1, keepdims=True)).astype(
                        jnp.bfloat16)
                    ctx_parts.append(lax.dot_general(
                        w, vg[:, hh * DH:(hh + 1) * DH],
                        (((1,), (0,)), ((), ())),
                        preferred_element_type=jnp.float32).astype(
                            jnp.bfloat16))
                ctx = jnp.concatenate(ctx_parts, axis=1)
                accs[b] = accs[b] + lax.dot_general(
                    ctx, wob[slot], (((1,), (0,)), ((), ())),
                    preferred_element_type=jnp.float32)

        for h in range(N_DEV - 1):
            rq = pltpu.make_async_remote_copy(
                src_ref=wqb.at[h], dst_ref=wqb.at[h + 1],
                send_sem=qsend.at[h], recv_sem=qrecv.at[h],
                device_id=(right,), device_id_type=pl.DeviceIdType.MESH)
            ro = pltpu.make_async_remote_copy(
                src_ref=wob.at[h], dst_ref=wob.at[h + 1],
                send_sem=osend.at[h], recv_sem=orecv.at[h],
                device_id=(right,), device_id_type=pl.DeviceIdType.MESH)
            rq.start()
            ro.start()
            compute_group(h, (me - h) % N_DEV)
            rq.wait()
            ro.wait()
        compute_group(N_DEV - 1, (me - (N_DEV - 1)) % N_DEV)

        for b in range(B):
            out_ref[b] = accs[b]

    return pl.pallas_call(
        body,
        out_shape=jax.ShapeDtypeStruct((B, SQ, D_MODEL), jnp.float32),
        in_specs=[pl.BlockSpec(memory_space=pltpu.VMEM)] * 5,
        out_specs=pl.BlockSpec(memory_space=pltpu.VMEM),
        scratch_shapes=[
            pltpu.VMEM((N_DEV, D_MODEL, G_COLS), jnp.bfloat16),
            pltpu.VMEM((N_DEV, G_COLS, D_MODEL), jnp.bfloat16),
            pltpu.SemaphoreType.DMA((N_DEV - 1,)),
            pltpu.SemaphoreType.DMA((N_DEV - 1,)),
            pltpu.SemaphoreType.DMA((N_DEV - 1,)),
            pltpu.SemaphoreType.DMA((N_DEV - 1,)),
            pltpu.SemaphoreType.DMA((2,)),
        ],
        compiler_params=pltpu.CompilerParams(collective_id=0),
    )(x, Wq, K2, V2, Wo)
